# baseline (speedup 1.0000x reference)
import functools
import math

import jax
import jax.numpy as jnp
from jax import lax
from jax.experimental import pallas as pl
from jax.experimental.pallas import tpu as pltpu
from jax.experimental.pallas import tpu_sc as plsc

N = 10000
E = 320000
D = 128
DE = 16
C = 128

NC = 2
NS = 16
NW = NC * NS
NP = 10240
SLC = NP // NS
CH = 128
NCHUNKS = E // CH
CPW = -(-NCHUNKS // NW)

_mesh = plsc.VectorSubcoreMesh(
    core_axis_name="c", subcore_axis_name="s", num_cores=NC, num_subcores=NS)
_sc_params = pltpu.CompilerParams(use_tc_tiling_on_sc=False,
                                  needs_layout_passes=False)

_NEG = -3.0e38


def _wid():
    return lax.axis_index("s") * NC + lax.axis_index("c")


def _fill_1d(ref, val):
    n = ref.shape[0]

    def body(i, _):
        ref[pl.ds(i * 16, 16)] = jnp.full((16,), val, ref.dtype)
        return 0

    lax.fori_loop(0, n // 16, body, 0)


def _fill_2d(ref, val):
    r, cc = ref.shape

    def body(i, _):
        for j in range(cc // 16):
            ref[i, pl.ds(j * 16, 16)] = jnp.full((16,), val, ref.dtype)
        return 0

    lax.fori_loop(0, r, body, 0)


def _merge32(part_hbm, tab_ref, tmp_ref, op):
    pltpu.sync_copy(part_hbm.at[0], tab_ref)

    def body(t, _):
        pltpu.sync_copy(part_hbm.at[t], tmp_ref)

        def inner(i, _):
            sl = pl.ds(i * 16, 16)
            tab_ref[sl] = op(tab_ref[sl], tmp_ref[sl])
            return 0

        lax.fori_loop(0, NP // 16, inner, 0)
        return 0

    lax.fori_loop(1, NW, body, 0)


def _p1_body(qext_hbm, k_hbm, ea_hbm, src_hbm, dst_hbm,
             alpha_hbm, amax_hbm,
             idx_s, idx_d, qrows, krows, ea_v, alpha_v,
             amax_priv, sem):
    _fill_1d(amax_priv, _NEG)
    w = _wid()

    def chunk(t, _):
        cid = w + NW * t

        @pl.when(cid < NCHUNKS)
        def _():
            base = cid * CH
            pltpu.sync_copy(src_hbm.at[pl.ds(base, CH)], idx_s)
            pltpu.sync_copy(dst_hbm.at[pl.ds(base, CH)], idx_d)
            pltpu.sync_copy(ea_hbm.at[pl.ds(base, CH)], ea_v)
            dq = pltpu.async_copy(qext_hbm.at[idx_d], qrows, sem)
            dq.wait()
            dk = pltpu.async_copy(k_hbm.at[idx_s], krows, sem)
            dk.wait()

            lane = lax.iota(jnp.int32, 16)

            def grp(g, _):
                acc = jnp.zeros((16,), jnp.float32)
                for l in range(16):
                    e = g * 16 + l
                    a16 = qrows[e, pl.ds(D, DE)] * ea_v[e, :]
                    for j in range(D // 16):
                        s16 = pl.ds(j * 16, 16)
                        a16 = a16 + qrows[e, s16] * krows[e, s16]
                    acc = jnp.where(lane == l, jnp.sum(a16), acc)
                sl = pl.ds(g * 16, 16)
                alpha_v[sl] = acc
                dv = idx_d[sl]

                def retry(cs):
                    i, _ = cs
                    cur = plsc.load_gather(amax_priv, [dv])
                    plsc.store_scatter(amax_priv, [dv], jnp.maximum(cur, acc))
                    chk = plsc.load_gather(amax_priv, [dv])
                    return i + 1, jnp.any(chk < acc)

                lax.while_loop(lambda cs: jnp.logical_and(cs[1], cs[0] < 16),
                               retry, (jnp.int32(0), jnp.bool_(True)))
                return 0

            lax.fori_loop(0, CH // 16, grp, 0)
            pltpu.sync_copy(alpha_v, alpha_hbm.at[pl.ds(base, CH)])

        return 0

    lax.fori_loop(0, CPW, chunk, 0)
    pltpu.sync_copy(amax_priv, amax_hbm.at[w])


def _p2_body(alpha_hbm, dst_hbm, amax_hbm,
             ex_hbm, denom_hbm,
             amax_tab, tmp_tab, denom_priv, idx_d, alpha_v, ex_v, sem):
    w = _wid()
    _merge32(amax_hbm, amax_tab, tmp_tab, jnp.maximum)
    _fill_1d(denom_priv, 0.0)

    def chunk(t, _):
        cid = w + NW * t

        @pl.when(cid < NCHUNKS)
        def _():
            base = cid * CH
            pltpu.sync_copy(alpha_hbm.at[pl.ds(base, CH)], alpha_v)
            pltpu.sync_copy(dst_hbm.at[pl.ds(base, CH)], idx_d)

            def grp(g, _):
                sl = pl.ds(g * 16, 16)
                dv = idx_d[sl]
                mx = plsc.load_gather(amax_tab, [dv])
                exv = jnp.exp(alpha_v[sl] - mx)
                ex_v[sl] = exv
                plsc.addupdate_scatter(denom_priv, [dv], exv)
                return 0

            lax.fori_loop(0, CH // 16, grp, 0)
            pltpu.sync_copy(ex_v, ex_hbm.at[pl.ds(base, CH)])

        return 0

    lax.fori_loop(0, CPW, chunk, 0)
    pltpu.sync_copy(denom_priv, denom_hbm.at[w])


def _merge32_chunked(part_hbm, tab_ref, tmp_ref, op):
    pltpu.sync_copy(part_hbm.at[0], tab_ref)

    def body(t, _):
        def band(bb, _):
            pltpu.sync_copy(part_hbm.at[t, pl.ds(bb * SLC, SLC)], tmp_ref)

            def inner(i, _):
                sl = pl.ds(bb * SLC + i * 16, 16)
                tab_ref[sl] = op(tab_ref[sl], tmp_ref[pl.ds(i * 16, 16)])
                return 0

            lax.fori_loop(0, SLC // 16, inner, 0)
            return 0

        lax.fori_loop(0, NP // SLC, band, 0)
        return 0

    lax.fori_loop(1, NW, body, 0)


def _p3_body(ex_hbm, src_hbm, dst_hbm, denom_hbm, v_hbm, ea_hbm,
             outp_hbm, sp_hbm,
             denom_tab, tmp_tab, idx_s, idx_d, attn_v, vrows, ea_v,
             zbuf, zbufs, outacc, sacc, sem):
    c = lax.axis_index("c")
    s = lax.axis_index("s")
    _merge32_chunked(denom_hbm, denom_tab, tmp_tab, jnp.add)
    _fill_2d(zbuf, 0.0)
    _fill_2d(zbufs, 0.0)
    for i in range(SLC // 32):
        rs = pl.ds(s * SLC + i * 32, 32)
        pltpu.sync_copy(zbuf, outacc.at[rs])
        pltpu.sync_copy(zbufs, sacc.at[rs])
    plsc.subcore_barrier()
    w = _wid()

    def chunk(t, _):
        cid = w + NW * t

        @pl.when(cid < NCHUNKS)
        def _():
            base = cid * CH
            pltpu.sync_copy(ex_hbm.at[pl.ds(base, CH)], attn_v)
            pltpu.sync_copy(src_hbm.at[pl.ds(base, CH)], idx_s)
            pltpu.sync_copy(dst_hbm.at[pl.ds(base, CH)], idx_d)
            pltpu.sync_copy(ea_hbm.at[pl.ds(base, CH)], ea_v)
            dv = pltpu.async_copy(v_hbm.at[idx_s], vrows, sem)
            dv.wait()

            def grp(g, _):
                sl16 = pl.ds(g * 16, 16)
                den = plsc.load_gather(denom_tab, [idx_d[sl16]])
                attnv = attn_v[sl16] / (den + 1e-16)
                for l in range(16):
                    e = g * 16 + l
                    a = attnv[l]
                    for j in range(D // 16):
                        s16 = pl.ds(j * 16, 16)
                        vrows[e, s16] = vrows[e, s16] * a
                    ea_v[e, :] = ea_v[e, :] * a
                return 0

            lax.fori_loop(0, CH // 16, grp, 0)
            pltpu.sync_copy(vrows, outacc.at[idx_d], add=True)
            pltpu.sync_copy(ea_v, sacc.at[idx_d], add=True)

        return 0

    lax.fori_loop(0, CPW, chunk, 0)
    plsc.subcore_barrier()
    rs = pl.ds(s * SLC, SLC)
    pltpu.sync_copy(outacc.at[rs], outp_hbm.at[c, rs])
    pltpu.sync_copy(sacc.at[rs], sp_hbm.at[c, rs])


def _p4_body(a_hbm, b_hbm, src_hbm, dst_hbm, wp2_hbm, bp2_hbm,
             pred_hbm,
             idx_s, idx_d, arows, brows, out_v, wp2_v, bp2_v, sem):
    pltpu.sync_copy(wp2_hbm, wp2_v)
    pltpu.sync_copy(bp2_hbm, bp2_v)
    w = _wid()

    def chunk(t, _):
        cid = w + NW * t

        @pl.when(cid < NCHUNKS)
        def _():
            base = cid * CH
            pltpu.sync_copy(src_hbm.at[pl.ds(base, CH)], idx_s)
            pltpu.sync_copy(dst_hbm.at[pl.ds(base, CH)], idx_d)
            da = pltpu.async_copy(a_hbm.at[idx_s], arows, sem)
            da.wait()
            db = pltpu.async_copy(b_hbm.at[idx_d], brows, sem)
            db.wait()
            bias = bp2_v[pl.ds(0, 16)][0]
            wp = [wp2_v[pl.ds(i * 16, 16)] for i in range(D // 16)]

            lane = lax.iota(jnp.int32, 16)

            def grp(g, _):
                z = jnp.zeros((16,), jnp.float32)
                for l in range(16):
                    e = g * 16 + l
                    a16 = jnp.zeros((16,), jnp.float32)
                    for j in range(D // 16):
                        s16 = pl.ds(j * 16, 16)
                        h = jnp.maximum(arows[e, s16] + brows[e, s16], 0.0)
                        a16 = a16 + h * wp[j]
                    z = jnp.where(lane == l, jnp.sum(a16), z)
                z = z + bias
                out_v[pl.ds(g * 16, 16)] = 1.0 / (1.0 + jnp.exp(-z))
                return 0

            lax.fori_loop(0, CH // 16, grp, 0)
            pltpu.sync_copy(out_v, pred_hbm.at[pl.ds(base, CH)])

        return 0

    lax.fori_loop(0, CPW, chunk, 0)


_BR = 1000


def _tables_body(x_ref, wq, bq, wk, bk, wv, bv, we, ws, bs,
                 q_o, qe_o, k_o, v_o, skip_o):
    x = x_ref[...]
    q = (jnp.dot(x, wq[...], preferred_element_type=jnp.float32) + bq[...]) \
        * (1.0 / math.sqrt(C))
    q_o[...] = q
    qe_o[...] = lax.dot_general(q, we[...], (((1,), (1,)), ((), ())),
                                preferred_element_type=jnp.float32)
    k_o[...] = jnp.dot(x, wk[...], preferred_element_type=jnp.float32) + bk[...]
    v_o[...] = jnp.dot(x, wv[...], preferred_element_type=jnp.float32) + bv[...]
    skip_o[...] = jnp.dot(x, ws[...], preferred_element_type=jnp.float32) + bs[...]


def _combine_body(outp_ref, sp_ref, skip_ref, wed_ref, x_o):
    o = outp_ref[0] + outp_ref[1]
    sacc = sp_ref[0] + sp_ref[1]
    x_o[...] = o + jnp.dot(sacc, wed_ref[...],
                           preferred_element_type=jnp.float32) + skip_ref[...]


def _pred_tables_body(x2_ref, wp1_ref, bp1_ref, a_o, b_o):
    x2 = x2_ref[...]
    wp1 = wp1_ref[...]
    a_o[...] = jnp.dot(x2, wp1[0:C, :],
                       preferred_element_type=jnp.float32) + bp1_ref[...]
    b_o[...] = jnp.dot(x2, wp1[C:2 * C, :], preferred_element_type=jnp.float32)


def _full(shape):
    return pl.BlockSpec(shape, lambda i: tuple(0 for _ in shape))


def _tc_tables(x, wq, bq, wk, bk, wv, bv, we, ws, bs):
    f = jnp.float32
    return pl.pallas_call(
        _tables_body,
        grid=(N // _BR,),
        in_specs=[
            pl.BlockSpec((_BR, D), lambda i: (i, 0)),
            _full((D, C)), _full((1, C)),
            _full((D, C)), _full((1, C)),
            _full((D, C)), _full((1, C)),
            _full((DE, C)),
            _full((D, C)), _full((1, C)),
        ],
        out_specs=[
            pl.BlockSpec((_BR, C), lambda i: (i, 0)),
            pl.BlockSpec((_BR, DE), lambda i: (i, 0)),
            pl.BlockSpec((_BR, C), lambda i: (i, 0)),
            pl.BlockSpec((_BR, C), lambda i: (i, 0)),
            pl.BlockSpec((_BR, C), lambda i: (i, 0)),
        ],
        out_shape=[
            jax.ShapeDtypeStruct((N, C), f),
            jax.ShapeDtypeStruct((N, DE), f),
            jax.ShapeDtypeStruct((N, C), f),
            jax.ShapeDtypeStruct((N, C), f),
            jax.ShapeDtypeStruct((N, C), f),
        ],
    )(x, wq, bq, wk, bk, wv, bv, we, ws, bs)


def _tc_combine(outp, sp, skip, we_folded):
    return pl.pallas_call(
        _combine_body,
        grid=(N // _BR,),
        in_specs=[
            pl.BlockSpec((2, _BR, C), lambda i: (0, i, 0)),
            pl.BlockSpec((2, _BR, DE), lambda i: (0, i, 0)),
            pl.BlockSpec((_BR, C), lambda i: (i, 0)),
            _full((DE, C)),
        ],
        out_specs=pl.BlockSpec((_BR, C), lambda i: (i, 0)),
        out_shape=jax.ShapeDtypeStruct((N, C), jnp.float32),
    )(outp, sp, skip, we_folded)


def _tc_pred_tables(x2, wp1, bp1):
    return pl.pallas_call(
        _pred_tables_body,
        grid=(N // _BR,),
        in_specs=[
            pl.BlockSpec((_BR, C), lambda i: (i, 0)),
            _full((2 * C, C)),
            _full((1, C)),
        ],
        out_specs=[
            pl.BlockSpec((_BR, C), lambda i: (i, 0)),
            pl.BlockSpec((_BR, C), lambda i: (i, 0)),
        ],
        out_shape=[
            jax.ShapeDtypeStruct((N, C), jnp.float32),
            jax.ShapeDtypeStruct((N, C), jnp.float32),
        ],
    )(x2, wp1, bp1)


def _sc_p1(qext, k, ea, src, dst):
    f = jnp.float32
    return pl.kernel(
        _p1_body,
        out_type=[jax.ShapeDtypeStruct((E,), f),
                  jax.ShapeDtypeStruct((NW, NP), f)],
        mesh=_mesh,
        compiler_params=_sc_params,
        scratch_types=[
            pltpu.VMEM((CH,), jnp.int32),
            pltpu.VMEM((CH,), jnp.int32),
            pltpu.VMEM((CH, D + DE), f),
            pltpu.VMEM((CH, D), f),
            pltpu.VMEM((CH, DE), f),
            pltpu.VMEM((CH,), f),
            pltpu.VMEM((NP,), f),
            pltpu.SemaphoreType.DMA,
        ],
    )(qext, k, ea, src, dst)


def _sc_p2(alpha, dst, amax):
    f = jnp.float32
    return pl.kernel(
        _p2_body,
        out_type=[jax.ShapeDtypeStruct((E,), f),
                  jax.ShapeDtypeStruct((NW, NP), f)],
        mesh=_mesh,
        compiler_params=_sc_params,
        scratch_types=[
            pltpu.VMEM((NP,), f),
            pltpu.VMEM((NP,), f),
            pltpu.VMEM((NP,), f),
            pltpu.VMEM((CH,), jnp.int32),
            pltpu.VMEM((CH,), f),
            pltpu.VMEM((CH,), f),
            pltpu.SemaphoreType.DMA,
        ],
    )(alpha, dst, amax)


def _sc_p3(ex, src, dst, denom, v, ea):
    f = jnp.float32
    return pl.kernel(
        _p3_body,
        out_type=[jax.ShapeDtypeStruct((NC, NP, C), f),
                  jax.ShapeDtypeStruct((NC, NP, DE), f)],
        mesh=_mesh,
        compiler_params=_sc_params,
        scratch_types=[
            pltpu.VMEM((NP,), f),
            pltpu.VMEM((SLC,), f),
            pltpu.VMEM((CH,), jnp.int32),
            pltpu.VMEM((CH,), jnp.int32),
            pltpu.VMEM((CH,), f),
            pltpu.VMEM((CH, C), f),
            pltpu.VMEM((CH, DE), f),
            pltpu.VMEM((32, C), f),
            pltpu.VMEM((32, DE), f),
            pltpu.VMEM_SHARED((NP, C), f),
            pltpu.VMEM_SHARED((NP, DE), f),
            pltpu.SemaphoreType.DMA,
        ],
    )(ex, src, dst, denom, v, ea)


def _sc_p4(a_tab, b_tab, src, dst, wp2, bp2):
    f = jnp.float32
    return pl.kernel(
        _p4_body,
        out_type=jax.ShapeDtypeStruct((E,), f),
        mesh=_mesh,
        compiler_params=_sc_params,
        scratch_types=[
            pltpu.VMEM((CH,), jnp.int32),
            pltpu.VMEM((CH,), jnp.int32),
            pltpu.VMEM((CH, C), f),
            pltpu.VMEM((CH, C), f),
            pltpu.VMEM((CH,), f),
            pltpu.VMEM((C,), f),
            pltpu.VMEM((16,), f),
            pltpu.SemaphoreType.DMA,
        ],
    )(a_tab, b_tab, src, dst, wp2, bp2)


def _layer(x_tabs, ea, src, dst, we_folded):
    qs, qe, k, v, skip = x_tabs
    qext = jnp.concatenate([qs, qe], axis=1)
    alpha, amax = _sc_p1(qext, k, ea, src, dst)
    ex, denom = _sc_p2(alpha, dst, amax)
    outp, sp = _sc_p3(ex, src, dst, denom, v, ea)
    return _tc_combine(outp[:, :N, :], sp[:, :N, :], skip, we_folded)


def kernel(x, edge_index, edge_attr,
           Wq1, bq1, Wk1, bk1, Wv1, bv1, We1, Ws1, bs1,
           Wq2, bq2, Wk2, bk2, Wv2, bv2, We2, Ws2, bs2,
           Wp1, bp1, Wp2, bp2):
    f = jnp.float32
    src = edge_index[0]
    dst = edge_index[1]
    m2 = jnp.dot(We1, We2)

    tabs1 = _tc_tables(x, Wq1, bq1.reshape(1, C), Wk1, bk1.reshape(1, C),
                       Wv1, bv1.reshape(1, C), We1, Ws1, bs1.reshape(1, C))
    x1 = _layer(tabs1, edge_attr, src, dst, We1)

    tabs2 = _tc_tables(x1, Wq2, bq2.reshape(1, C), Wk2, bk2.reshape(1, C),
                       Wv2, bv2.reshape(1, C), m2, Ws2, bs2.reshape(1, C))
    x2 = _layer(tabs2, edge_attr, src, dst, m2)

    a_tab, b_tab = _tc_pred_tables(x2, Wp1, bp1.reshape(1, C))
    wp2pad = jnp.pad(bp2.astype(f), (0, 15))
    return _sc_p4(a_tab, b_tab, src, dst, Wp2.reshape(C), wp2pad)


# speedup vs baseline: 1.3379x; 1.3379x over previous
import functools
import math

import jax
import jax.numpy as jnp
from jax import lax
from jax.experimental import pallas as pl
from jax.experimental.pallas import tpu as pltpu
from jax.experimental.pallas import tpu_sc as plsc

N = 10000
E = 320000
D = 128
DE = 16
C = 128

NC = 2
NS = 16
NW = NC * NS
NP = 10240
SLC = NP // NS
CH = 128
NCHUNKS = E // CH
CPW = -(-NCHUNKS // NW)

_mesh = plsc.VectorSubcoreMesh(
    core_axis_name="c", subcore_axis_name="s", num_cores=NC, num_subcores=NS)
_sc_params = pltpu.CompilerParams(use_tc_tiling_on_sc=False,
                                  needs_layout_passes=False)

_NEG = -3.0e38


def _wid():
    return lax.axis_index("s") * NC + lax.axis_index("c")


def _fill_1d(ref, val):
    n = ref.shape[0]

    def body(i, _):
        ref[pl.ds(i * 16, 16)] = jnp.full((16,), val, ref.dtype)
        return 0

    lax.fori_loop(0, n // 16, body, 0)


def _fill_2d(ref, val):
    r, cc = ref.shape

    def body(i, _):
        for j in range(cc // 16):
            ref[i, pl.ds(j * 16, 16)] = jnp.full((16,), val, ref.dtype)
        return 0

    lax.fori_loop(0, r, body, 0)


def _merge32(part_hbm, tab_ref, tmp_ref, op):
    pltpu.sync_copy(part_hbm.at[0], tab_ref)

    def body(t, _):
        pltpu.sync_copy(part_hbm.at[t], tmp_ref)

        def inner(i, _):
            sl = pl.ds(i * 16, 16)
            tab_ref[sl] = op(tab_ref[sl], tmp_ref[sl])
            return 0

        lax.fori_loop(0, NP // 16, inner, 0)
        return 0

    lax.fori_loop(1, NW, body, 0)


def _p1_body(qext_hbm, k_hbm, ea_hbm, src_hbm, dst_hbm,
             alpha_hbm, amax_hbm,
             is0, id0, ea0, q0, k0, is1, id1, ea1, q1, k1,
             alpha_v, amax_priv, si0, si1, sg0, sg1):
    bufs = ((is0, id0, ea0, q0, k0, si0, sg0),
            (is1, id1, ea1, q1, k1, si1, sg1))
    _fill_1d(amax_priv, _NEG)
    w = _wid()
    lane = lax.iota(jnp.int32, 16)

    def valid(t):
        return (w + NW * t) < NCHUNKS

    def issue_idx(b, t):
        bs, bd, bea, _, _, si, _ = bufs[b]
        base = (w + NW * t) * CH

        @pl.when(valid(t))
        def _():
            pltpu.async_copy(src_hbm.at[pl.ds(base, CH)], bs, si)
            pltpu.async_copy(dst_hbm.at[pl.ds(base, CH)], bd, si)
            pltpu.async_copy(ea_hbm.at[pl.ds(base, CH)], bea, si)

    def wait_idx(b, t):
        bs, bd, bea, _, _, si, _ = bufs[b]

        @pl.when(valid(t))
        def _():
            pltpu.make_async_copy(src_hbm.at[pl.ds(0, CH)], bs, si).wait()
            pltpu.make_async_copy(dst_hbm.at[pl.ds(0, CH)], bd, si).wait()
            pltpu.make_async_copy(ea_hbm.at[pl.ds(0, CH)], bea, si).wait()

    def issue_gather(b, t):
        bs, bd, _, bq, bk, _, sg = bufs[b]

        @pl.when(valid(t))
        def _():
            pltpu.async_copy(qext_hbm.at[bd], bq, sg)
            pltpu.async_copy(k_hbm.at[bs], bk, sg)

    def wait_gather(b, t):
        _, _, _, bq, bk, _, sg = bufs[b]

        @pl.when(valid(t))
        def _():
            pltpu.make_async_copy(qext_hbm.at[pl.ds(0, CH)], bq, sg).wait()
            pltpu.make_async_copy(k_hbm.at[pl.ds(0, CH)], bk, sg).wait()

    def compute(b, t):
        bs, bd, bea, bq, bk, _, _ = bufs[b]

        @pl.when(valid(t))
        def _():
            base = (w + NW * t) * CH

            def grp(g, _):
                acc = jnp.zeros((16,), jnp.float32)
                for l in range(16):
                    e = g * 16 + l
                    a16 = bq[e, pl.ds(D, DE)] * bea[e, :]
                    for j in range(D // 16):
                        s16 = pl.ds(j * 16, 16)
                        a16 = a16 + bq[e, s16] * bk[e, s16]
                    acc = jnp.where(lane == l, jnp.sum(a16), acc)
                sl = pl.ds(g * 16, 16)
                alpha_v[sl] = acc
                dv = bd[sl]

                def retry(cs):
                    i, _ = cs
                    cur = plsc.load_gather(amax_priv, [dv])
                    plsc.store_scatter(amax_priv, [dv], jnp.maximum(cur, acc))
                    chk = plsc.load_gather(amax_priv, [dv])
                    return i + 1, jnp.any(chk < acc)

                lax.while_loop(lambda cs: jnp.logical_and(cs[1], cs[0] < 16),
                               retry, (jnp.int32(0), jnp.bool_(True)))
                return 0

            lax.fori_loop(0, CH // 16, grp, 0)
            pltpu.sync_copy(alpha_v, alpha_hbm.at[pl.ds(base, CH)])

    issue_idx(0, 0)
    wait_idx(0, 0)
    issue_gather(0, 0)
    issue_idx(1, 1)

    def outer(t2, _):
        for b in (0, 1):
            t = t2 * 2 + b
            nb = 1 - b
            wait_idx(nb, t + 1)
            issue_gather(nb, t + 1)
            wait_gather(b, t)
            compute(b, t)
            issue_idx(b, t + 2)
        return 0

    lax.fori_loop(0, (CPW + 1) // 2, outer, 0)
    pltpu.sync_copy(amax_priv, amax_hbm.at[w])


def _p2_body(alpha_hbm, dst_hbm, amax_hbm,
             ex_hbm, denom_hbm,
             amax_tab, tmp_tab, denom_priv, idx_d, alpha_v, ex_v, sem):
    w = _wid()
    _merge32(amax_hbm, amax_tab, tmp_tab, jnp.maximum)
    _fill_1d(denom_priv, 0.0)

    def chunk(t, _):
        cid = w + NW * t

        @pl.when(cid < NCHUNKS)
        def _():
            base = cid * CH
            pltpu.sync_copy(alpha_hbm.at[pl.ds(base, CH)], alpha_v)
            pltpu.sync_copy(dst_hbm.at[pl.ds(base, CH)], idx_d)

            def grp(g, _):
                sl = pl.ds(g * 16, 16)
                dv = idx_d[sl]
                mx = plsc.load_gather(amax_tab, [dv])
                exv = jnp.exp(alpha_v[sl] - mx)
                ex_v[sl] = exv
                plsc.addupdate_scatter(denom_priv, [dv], exv)
                return 0

            lax.fori_loop(0, CH // 16, grp, 0)
            pltpu.sync_copy(ex_v, ex_hbm.at[pl.ds(base, CH)])

        return 0

    lax.fori_loop(0, CPW, chunk, 0)
    pltpu.sync_copy(denom_priv, denom_hbm.at[w])


def _p2b_body(ex_hbm, dst_hbm, denom_hbm,
              attn_hbm,
              denom_tab, tmp_tab, idx_d, ex_v, sem):
    w = _wid()
    _merge32(denom_hbm, denom_tab, tmp_tab, jnp.add)

    def chunk(t, _):
        cid = w + NW * t

        @pl.when(cid < NCHUNKS)
        def _():
            base = cid * CH
            pltpu.sync_copy(ex_hbm.at[pl.ds(base, CH)], ex_v)
            pltpu.sync_copy(dst_hbm.at[pl.ds(base, CH)], idx_d)

            def grp(g, _):
                sl = pl.ds(g * 16, 16)
                dv = idx_d[sl]
                den = plsc.load_gather(denom_tab, [dv])
                ex_v[sl] = ex_v[sl] / (den + 1e-16)
                return 0

            lax.fori_loop(0, CH // 16, grp, 0)
            pltpu.sync_copy(ex_v, attn_hbm.at[pl.ds(base, CH)])

        return 0

    lax.fori_loop(0, CPW, chunk, 0)


def _p3_body(attn_hbm, src_hbm, dst_hbm, v_hbm, ea_hbm,
             outp_hbm, sp_hbm,
             idx_s, idx_d, attn_v, vrows, ea_v, zbuf, zbufs, outacc, sacc, sem):
    c = lax.axis_index("c")
    s = lax.axis_index("s")
    _fill_2d(zbuf, 0.0)
    _fill_2d(zbufs, 0.0)
    for i in range(SLC // 32):
        rs = pl.ds(s * SLC + i * 32, 32)
        pltpu.sync_copy(zbuf, outacc.at[rs])
        pltpu.sync_copy(zbufs, sacc.at[rs])
    plsc.subcore_barrier()
    w = _wid()

    def chunk(t, _):
        cid = w + NW * t

        @pl.when(cid < NCHUNKS)
        def _():
            base = cid * CH
            pltpu.sync_copy(attn_hbm.at[pl.ds(base, CH)], attn_v)
            pltpu.sync_copy(src_hbm.at[pl.ds(base, CH)], idx_s)
            pltpu.sync_copy(dst_hbm.at[pl.ds(base, CH)], idx_d)
            pltpu.sync_copy(ea_hbm.at[pl.ds(base, CH)], ea_v)
            dv = pltpu.async_copy(v_hbm.at[idx_s], vrows, sem)
            dv.wait()

            def grp(g, _):
                attnv = attn_v[pl.ds(g * 16, 16)]
                for l in range(16):
                    e = g * 16 + l
                    a = attnv[l]
                    for j in range(D // 16):
                        s16 = pl.ds(j * 16, 16)
                        vrows[e, s16] = vrows[e, s16] * a
                    ea_v[e, :] = ea_v[e, :] * a
                return 0

            lax.fori_loop(0, CH // 16, grp, 0)
            pltpu.sync_copy(vrows, outacc.at[idx_d], add=True)
            pltpu.sync_copy(ea_v, sacc.at[idx_d], add=True)

        return 0

    lax.fori_loop(0, CPW, chunk, 0)
    plsc.subcore_barrier()
    rs = pl.ds(s * SLC, SLC)
    pltpu.sync_copy(outacc.at[rs], outp_hbm.at[c, rs])
    pltpu.sync_copy(sacc.at[rs], sp_hbm.at[c, rs])


def _p4_body(a_hbm, b_hbm, src_hbm, dst_hbm, wp2_hbm, bp2_hbm,
             pred_hbm,
             is0, id0, ar0, br0, is1, id1, ar1, br1,
             out_v, wp2_v, bp2_v, si0, si1, sg0, sg1):
    bufs = ((is0, id0, ar0, br0, si0, sg0),
            (is1, id1, ar1, br1, si1, sg1))
    pltpu.sync_copy(wp2_hbm, wp2_v)
    pltpu.sync_copy(bp2_hbm, bp2_v)
    w = _wid()
    lane = lax.iota(jnp.int32, 16)
    bias = bp2_v[pl.ds(0, 16)][0]
    wp = [wp2_v[pl.ds(i * 16, 16)] for i in range(D // 16)]

    def valid(t):
        return (w + NW * t) < NCHUNKS

    def issue_idx(b, t):
        bs, bd, _, _, si, _ = bufs[b]
        base = (w + NW * t) * CH

        @pl.when(valid(t))
        def _():
            pltpu.async_copy(src_hbm.at[pl.ds(base, CH)], bs, si)
            pltpu.async_copy(dst_hbm.at[pl.ds(base, CH)], bd, si)

    def wait_idx(b, t):
        bs, bd, _, _, si, _ = bufs[b]

        @pl.when(valid(t))
        def _():
            pltpu.make_async_copy(src_hbm.at[pl.ds(0, CH)], bs, si).wait()
            pltpu.make_async_copy(dst_hbm.at[pl.ds(0, CH)], bd, si).wait()

    def issue_gather(b, t):
        bs, bd, ba, bb, _, sg = bufs[b]

        @pl.when(valid(t))
        def _():
            pltpu.async_copy(a_hbm.at[bs], ba, sg)
            pltpu.async_copy(b_hbm.at[bd], bb, sg)

    def wait_gather(b, t):
        _, _, ba, bb, _, sg = bufs[b]

        @pl.when(valid(t))
        def _():
            pltpu.make_async_copy(a_hbm.at[pl.ds(0, CH)], ba, sg).wait()
            pltpu.make_async_copy(b_hbm.at[pl.ds(0, CH)], bb, sg).wait()

    def compute(b, t):
        _, _, ba, bb, _, _ = bufs[b]

        @pl.when(valid(t))
        def _():
            base = (w + NW * t) * CH

            def grp(g, _):
                z = jnp.zeros((16,), jnp.float32)
                for l in range(16):
                    e = g * 16 + l
                    a16 = jnp.zeros((16,), jnp.float32)
                    for j in range(D // 16):
                        s16 = pl.ds(j * 16, 16)
                        h = jnp.maximum(ba[e, s16] + bb[e, s16], 0.0)
                        a16 = a16 + h * wp[j]
                    z = jnp.where(lane == l, jnp.sum(a16), z)
                z = z + bias
                out_v[pl.ds(g * 16, 16)] = 1.0 / (1.0 + jnp.exp(-z))
                return 0

            lax.fori_loop(0, CH // 16, grp, 0)
            pltpu.sync_copy(out_v, pred_hbm.at[pl.ds(base, CH)])

    issue_idx(0, 0)
    wait_idx(0, 0)
    issue_gather(0, 0)
    issue_idx(1, 1)

    def outer(t2, _):
        for b in (0, 1):
            t = t2 * 2 + b
            nb = 1 - b
            wait_idx(nb, t + 1)
            issue_gather(nb, t + 1)
            wait_gather(b, t)
            compute(b, t)
            issue_idx(b, t + 2)
        return 0

    lax.fori_loop(0, (CPW + 1) // 2, outer, 0)


_BR = 1000


def _tables_body(x_ref, wq, bq, wk, bk, wv, bv, we, ws, bs,
                 q_o, qe_o, k_o, v_o, skip_o):
    x = x_ref[...]
    q = (jnp.dot(x, wq[...], preferred_element_type=jnp.float32) + bq[...]) \
        * (1.0 / math.sqrt(C))
    q_o[...] = q
    qe_o[...] = lax.dot_general(q, we[...], (((1,), (1,)), ((), ())),
                                preferred_element_type=jnp.float32)
    k_o[...] = jnp.dot(x, wk[...], preferred_element_type=jnp.float32) + bk[...]
    v_o[...] = jnp.dot(x, wv[...], preferred_element_type=jnp.float32) + bv[...]
    skip_o[...] = jnp.dot(x, ws[...], preferred_element_type=jnp.float32) + bs[...]


def _combine_body(outp_ref, sp_ref, skip_ref, wed_ref, x_o):
    o = outp_ref[0] + outp_ref[1]
    sacc = sp_ref[0] + sp_ref[1]
    x_o[...] = o + jnp.dot(sacc, wed_ref[...],
                           preferred_element_type=jnp.float32) + skip_ref[...]


def _pred_tables_body(x2_ref, wp1_ref, bp1_ref, a_o, b_o):
    x2 = x2_ref[...]
    wp1 = wp1_ref[...]
    a_o[...] = jnp.dot(x2, wp1[0:C, :],
                       preferred_element_type=jnp.float32) + bp1_ref[...]
    b_o[...] = jnp.dot(x2, wp1[C:2 * C, :], preferred_element_type=jnp.float32)


def _full(shape):
    return pl.BlockSpec(shape, lambda i: tuple(0 for _ in shape))


def _tc_tables(x, wq, bq, wk, bk, wv, bv, we, ws, bs):
    f = jnp.float32
    return pl.pallas_call(
        _tables_body,
        grid=(N // _BR,),
        in_specs=[
            pl.BlockSpec((_BR, D), lambda i: (i, 0)),
            _full((D, C)), _full((1, C)),
            _full((D, C)), _full((1, C)),
            _full((D, C)), _full((1, C)),
            _full((DE, C)),
            _full((D, C)), _full((1, C)),
        ],
        out_specs=[
            pl.BlockSpec((_BR, C), lambda i: (i, 0)),
            pl.BlockSpec((_BR, DE), lambda i: (i, 0)),
            pl.BlockSpec((_BR, C), lambda i: (i, 0)),
            pl.BlockSpec((_BR, C), lambda i: (i, 0)),
            pl.BlockSpec((_BR, C), lambda i: (i, 0)),
        ],
        out_shape=[
            jax.ShapeDtypeStruct((N, C), f),
            jax.ShapeDtypeStruct((N, DE), f),
            jax.ShapeDtypeStruct((N, C), f),
            jax.ShapeDtypeStruct((N, C), f),
            jax.ShapeDtypeStruct((N, C), f),
        ],
    )(x, wq, bq, wk, bk, wv, bv, we, ws, bs)


def _tc_combine(outp, sp, skip, we_folded):
    return pl.pallas_call(
        _combine_body,
        grid=(N // _BR,),
        in_specs=[
            pl.BlockSpec((2, _BR, C), lambda i: (0, i, 0)),
            pl.BlockSpec((2, _BR, DE), lambda i: (0, i, 0)),
            pl.BlockSpec((_BR, C), lambda i: (i, 0)),
            _full((DE, C)),
        ],
        out_specs=pl.BlockSpec((_BR, C), lambda i: (i, 0)),
        out_shape=jax.ShapeDtypeStruct((N, C), jnp.float32),
    )(outp, sp, skip, we_folded)


def _tc_pred_tables(x2, wp1, bp1):
    return pl.pallas_call(
        _pred_tables_body,
        grid=(N // _BR,),
        in_specs=[
            pl.BlockSpec((_BR, C), lambda i: (i, 0)),
            _full((2 * C, C)),
            _full((1, C)),
        ],
        out_specs=[
            pl.BlockSpec((_BR, C), lambda i: (i, 0)),
            pl.BlockSpec((_BR, C), lambda i: (i, 0)),
        ],
        out_shape=[
            jax.ShapeDtypeStruct((N, C), jnp.float32),
            jax.ShapeDtypeStruct((N, C), jnp.float32),
        ],
    )(x2, wp1, bp1)


def _sc_p1(qext, k, ea, src, dst):
    f = jnp.float32
    return pl.kernel(
        _p1_body,
        out_type=[jax.ShapeDtypeStruct((E,), f),
                  jax.ShapeDtypeStruct((NW, NP), f)],
        mesh=_mesh,
        compiler_params=_sc_params,
        scratch_types=(
            [pltpu.VMEM((CH,), jnp.int32), pltpu.VMEM((CH,), jnp.int32),
             pltpu.VMEM((CH, DE), f), pltpu.VMEM((CH, D + DE), f),
             pltpu.VMEM((CH, D), f)] * 2 +
            [pltpu.VMEM((CH,), f), pltpu.VMEM((NP,), f)] +
            [pltpu.SemaphoreType.DMA] * 4),
    )(qext, k, ea, src, dst)


def _sc_p2(alpha, dst, amax):
    f = jnp.float32
    return pl.kernel(
        _p2_body,
        out_type=[jax.ShapeDtypeStruct((E,), f),
                  jax.ShapeDtypeStruct((NW, NP), f)],
        mesh=_mesh,
        compiler_params=_sc_params,
        scratch_types=[
            pltpu.VMEM((NP,), f),
            pltpu.VMEM((NP,), f),
            pltpu.VMEM((NP,), f),
            pltpu.VMEM((CH,), jnp.int32),
            pltpu.VMEM((CH,), f),
            pltpu.VMEM((CH,), f),
            pltpu.SemaphoreType.DMA,
        ],
    )(alpha, dst, amax)


def _sc_p2b(ex, dst, denom):
    f = jnp.float32
    return pl.kernel(
        _p2b_body,
        out_type=jax.ShapeDtypeStruct((E,), f),
        mesh=_mesh,
        compiler_params=_sc_params,
        scratch_types=[
            pltpu.VMEM((NP,), f),
            pltpu.VMEM((NP,), f),
            pltpu.VMEM((CH,), jnp.int32),
            pltpu.VMEM((CH,), f),
            pltpu.SemaphoreType.DMA,
        ],
    )(ex, dst, denom)


def _sc_p3(attn, src, dst, v, ea):
    f = jnp.float32
    return pl.kernel(
        _p3_body,
        out_type=[jax.ShapeDtypeStruct((NC, NP, C), f),
                  jax.ShapeDtypeStruct((NC, NP, DE), f)],
        mesh=_mesh,
        compiler_params=_sc_params,
        scratch_types=[
            pltpu.VMEM((CH,), jnp.int32),
            pltpu.VMEM((CH,), jnp.int32),
            pltpu.VMEM((CH,), f),
            pltpu.VMEM((CH, C), f),
            pltpu.VMEM((CH, DE), f),
            pltpu.VMEM((32, C), f),
            pltpu.VMEM((32, DE), f),
            pltpu.VMEM_SHARED((NP, C), f),
            pltpu.VMEM_SHARED((NP, DE), f),
            pltpu.SemaphoreType.DMA,
        ],
    )(attn, src, dst, v, ea)


def _sc_p4(a_tab, b_tab, src, dst, wp2, bp2):
    f = jnp.float32
    return pl.kernel(
        _p4_body,
        out_type=jax.ShapeDtypeStruct((E,), f),
        mesh=_mesh,
        compiler_params=_sc_params,
        scratch_types=(
            [pltpu.VMEM((CH,), jnp.int32), pltpu.VMEM((CH,), jnp.int32),
             pltpu.VMEM((CH, C), f), pltpu.VMEM((CH, C), f)] * 2 +
            [pltpu.VMEM((CH,), f), pltpu.VMEM((C,), f), pltpu.VMEM((16,), f)] +
            [pltpu.SemaphoreType.DMA] * 4),
    )(a_tab, b_tab, src, dst, wp2, bp2)


def _layer(x_tabs, ea, src, dst, we_folded):
    qs, qe, k, v, skip = x_tabs
    qext = jnp.concatenate([qs, qe], axis=1)
    alpha, amax = _sc_p1(qext, k, ea, src, dst)
    ex, denom = _sc_p2(alpha, dst, amax)
    attn = _sc_p2b(ex, dst, denom)
    outp, sp = _sc_p3(attn, src, dst, v, ea)
    return _tc_combine(outp[:, :N, :], sp[:, :N, :], skip, we_folded)


def kernel(x, edge_index, edge_attr,
           Wq1, bq1, Wk1, bk1, Wv1, bv1, We1, Ws1, bs1,
           Wq2, bq2, Wk2, bk2, Wv2, bv2, We2, Ws2, bs2,
           Wp1, bp1, Wp2, bp2):
    f = jnp.float32
    src = edge_index[0]
    dst = edge_index[1]
    m2 = jnp.dot(We1, We2)

    tabs1 = _tc_tables(x, Wq1, bq1.reshape(1, C), Wk1, bk1.reshape(1, C),
                       Wv1, bv1.reshape(1, C), We1, Ws1, bs1.reshape(1, C))
    x1 = _layer(tabs1, edge_attr, src, dst, We1)

    tabs2 = _tc_tables(x1, Wq2, bq2.reshape(1, C), Wk2, bk2.reshape(1, C),
                       Wv2, bv2.reshape(1, C), m2, Ws2, bs2.reshape(1, C))
    x2 = _layer(tabs2, edge_attr, src, dst, m2)

    a_tab, b_tab = _tc_pred_tables(x2, Wp1, bp1.reshape(1, C))
    wp2pad = jnp.pad(bp2.astype(f), (0, 15))
    return _sc_p4(a_tab, b_tab, src, dst, Wp2.reshape(C), wp2pad)


# speedup vs baseline: 1.5335x; 1.1462x over previous
import functools
import math

import jax
import jax.numpy as jnp
from jax import lax
from jax.experimental import pallas as pl
from jax.experimental.pallas import tpu as pltpu
from jax.experimental.pallas import tpu_sc as plsc

N = 10000
E = 320000
D = 128
DE = 16
C = 128

NC = 2
NS = 16
NW = NC * NS
NP = 10240
SLC = NP // NS
CH = 128
NCHUNKS = E // CH
CPW = -(-NCHUNKS // NW)

_mesh = plsc.VectorSubcoreMesh(
    core_axis_name="c", subcore_axis_name="s", num_cores=NC, num_subcores=NS)
_sc_params = pltpu.CompilerParams(use_tc_tiling_on_sc=False,
                                  needs_layout_passes=False)

_NEG = -3.0e38


def _wid():
    return lax.axis_index("s") * NC + lax.axis_index("c")


def _fill_1d(ref, val):
    n = ref.shape[0]

    def body(i, _):
        ref[pl.ds(i * 16, 16)] = jnp.full((16,), val, ref.dtype)
        return 0

    lax.fori_loop(0, n // 16, body, 0)


def _fill_2d(ref, val):
    r, cc = ref.shape

    def body(i, _):
        for j in range(cc // 16):
            ref[i, pl.ds(j * 16, 16)] = jnp.full((16,), val, ref.dtype)
        return 0

    lax.fori_loop(0, r, body, 0)


def _merge32(part_hbm, tab_ref, tmp_ref, op):
    pltpu.sync_copy(part_hbm.at[0], tab_ref)

    def body(t, _):
        pltpu.sync_copy(part_hbm.at[t], tmp_ref)

        def inner(i, _):
            sl = pl.ds(i * 16, 16)
            tab_ref[sl] = op(tab_ref[sl], tmp_ref[sl])
            return 0

        lax.fori_loop(0, NP // 16, inner, 0)
        return 0

    lax.fori_loop(1, NW, body, 0)


def _p1_body(qext_hbm, k_hbm, ea_hbm, src_hbm, dst_hbm,
             alpha_hbm, amax_hbm,
             is0, id0, ea0, q0, k0, is1, id1, ea1, q1, k1,
             alpha_v, amax_priv, si0, si1, sg0, sg1):
    bufs = ((is0, id0, ea0, q0, k0, si0, sg0),
            (is1, id1, ea1, q1, k1, si1, sg1))
    _fill_1d(amax_priv, _NEG)
    w = _wid()
    lane = lax.iota(jnp.int32, 16)

    def valid(t):
        return (w + NW * t) < NCHUNKS

    def issue_idx(b, t):
        bs, bd, bea, _, _, si, _ = bufs[b]
        base = (w + NW * t) * CH

        @pl.when(valid(t))
        def _():
            pltpu.async_copy(src_hbm.at[pl.ds(base, CH)], bs, si)
            pltpu.async_copy(dst_hbm.at[pl.ds(base, CH)], bd, si)
            pltpu.async_copy(ea_hbm.at[pl.ds(base, CH)], bea, si)

    def wait_idx(b, t):
        bs, bd, bea, _, _, si, _ = bufs[b]

        @pl.when(valid(t))
        def _():
            pltpu.make_async_copy(src_hbm.at[pl.ds(0, CH)], bs, si).wait()
            pltpu.make_async_copy(dst_hbm.at[pl.ds(0, CH)], bd, si).wait()
            pltpu.make_async_copy(ea_hbm.at[pl.ds(0, CH)], bea, si).wait()

    def issue_gather(b, t):
        bs, bd, _, bq, bk, _, sg = bufs[b]

        @pl.when(valid(t))
        def _():
            pltpu.async_copy(qext_hbm.at[bd], bq, sg)
            pltpu.async_copy(k_hbm.at[bs], bk, sg)

    def wait_gather(b, t):
        _, _, _, bq, bk, _, sg = bufs[b]

        @pl.when(valid(t))
        def _():
            pltpu.make_async_copy(qext_hbm.at[pl.ds(0, CH)], bq, sg).wait()
            pltpu.make_async_copy(k_hbm.at[pl.ds(0, CH)], bk, sg).wait()

    def compute(b, t):
        bs, bd, bea, bq, bk, _, _ = bufs[b]

        @pl.when(valid(t))
        def _():
            base = (w + NW * t) * CH

            def grp(g, _):
                acc = jnp.zeros((16,), jnp.float32)
                for l in range(16):
                    e = g * 16 + l
                    a16 = bq[e, pl.ds(D, DE)] * bea[e, :]
                    for j in range(D // 16):
                        s16 = pl.ds(j * 16, 16)
                        a16 = a16 + bq[e, s16] * bk[e, s16]
                    acc = jnp.where(lane == l, jnp.sum(a16), acc)
                sl = pl.ds(g * 16, 16)
                alpha_v[sl] = acc
                dv = bd[sl]

                def retry(cs):
                    i, _ = cs
                    cur = plsc.load_gather(amax_priv, [dv])
                    plsc.store_scatter(amax_priv, [dv], jnp.maximum(cur, acc))
                    chk = plsc.load_gather(amax_priv, [dv])
                    return i + 1, jnp.any(chk < acc)

                lax.while_loop(lambda cs: jnp.logical_and(cs[1], cs[0] < 16),
                               retry, (jnp.int32(0), jnp.bool_(True)))
                return 0

            lax.fori_loop(0, CH // 16, grp, 0)
            pltpu.sync_copy(alpha_v, alpha_hbm.at[pl.ds(base, CH)])

    issue_idx(0, 0)
    wait_idx(0, 0)
    issue_gather(0, 0)
    issue_idx(1, 1)

    def outer(t2, _):
        for b in (0, 1):
            t = t2 * 2 + b
            nb = 1 - b
            wait_idx(nb, t + 1)
            issue_gather(nb, t + 1)
            wait_gather(b, t)
            compute(b, t)
            issue_idx(b, t + 2)
        return 0

    lax.fori_loop(0, (CPW + 1) // 2, outer, 0)
    pltpu.sync_copy(amax_priv, amax_hbm.at[w])


def _p2_body(alpha_hbm, dst_hbm, amax_hbm,
             ex_hbm, denom_hbm,
             amax_tab, tmp_tab, denom_priv, idx_d, alpha_v, ex_v, sem):
    w = _wid()
    _merge32(amax_hbm, amax_tab, tmp_tab, jnp.maximum)
    _fill_1d(denom_priv, 0.0)

    def chunk(t, _):
        cid = w + NW * t

        @pl.when(cid < NCHUNKS)
        def _():
            base = cid * CH
            pltpu.sync_copy(alpha_hbm.at[pl.ds(base, CH)], alpha_v)
            pltpu.sync_copy(dst_hbm.at[pl.ds(base, CH)], idx_d)

            def grp(g, _):
                sl = pl.ds(g * 16, 16)
                dv = idx_d[sl]
                mx = plsc.load_gather(amax_tab, [dv])
                exv = jnp.exp(alpha_v[sl] - mx)
                ex_v[sl] = exv
                plsc.addupdate_scatter(denom_priv, [dv], exv)
                return 0

            lax.fori_loop(0, CH // 16, grp, 0)
            pltpu.sync_copy(ex_v, ex_hbm.at[pl.ds(base, CH)])

        return 0

    lax.fori_loop(0, CPW, chunk, 0)
    pltpu.sync_copy(denom_priv, denom_hbm.at[w])


def _p2b_body(ex_hbm, dst_hbm, denom_hbm,
              attn_hbm,
              denom_tab, tmp_tab, idx_d, ex_v, sem):
    w = _wid()
    _merge32(denom_hbm, denom_tab, tmp_tab, jnp.add)

    def chunk(t, _):
        cid = w + NW * t

        @pl.when(cid < NCHUNKS)
        def _():
            base = cid * CH
            pltpu.sync_copy(ex_hbm.at[pl.ds(base, CH)], ex_v)
            pltpu.sync_copy(dst_hbm.at[pl.ds(base, CH)], idx_d)

            def grp(g, _):
                sl = pl.ds(g * 16, 16)
                dv = idx_d[sl]
                den = plsc.load_gather(denom_tab, [dv])
                ex_v[sl] = ex_v[sl] / (den + 1e-16)
                return 0

            lax.fori_loop(0, CH // 16, grp, 0)
            pltpu.sync_copy(ex_v, attn_hbm.at[pl.ds(base, CH)])

        return 0

    lax.fori_loop(0, CPW, chunk, 0)


def _p3_body(attn_hbm, src_hbm, dst_hbm, v_hbm, ea_hbm,
             outp_hbm, sp_hbm,
             is0, id0, at0, ea0, vr0, is1, id1, at1, ea1, vr1,
             outacc, sacc, si0, si1, sg0, sg1):
    bufs = ((is0, id0, at0, ea0, vr0, si0, sg0),
            (is1, id1, at1, ea1, vr1, si1, sg1))
    c = lax.axis_index("c")
    s = lax.axis_index("s")
    w = _wid()
    _fill_2d(vr0, 0.0)
    _fill_2d(ea0, 0.0)
    for i in range(SLC // CH):
        rs = pl.ds(s * SLC + i * CH, CH)
        pltpu.sync_copy(vr0, outacc.at[rs])
        pltpu.sync_copy(ea0, sacc.at[rs])
    plsc.subcore_barrier()

    def valid(t):
        return (w + NW * t) < NCHUNKS

    def issue_idx(b, t):
        bs, bd, bat, bea, _, si, _ = bufs[b]
        base = (w + NW * t) * CH

        @pl.when(valid(t))
        def _():
            pltpu.async_copy(src_hbm.at[pl.ds(base, CH)], bs, si)
            pltpu.async_copy(dst_hbm.at[pl.ds(base, CH)], bd, si)
            pltpu.async_copy(attn_hbm.at[pl.ds(base, CH)], bat, si)
            pltpu.async_copy(ea_hbm.at[pl.ds(base, CH)], bea, si)

    def wait_idx(b, t):
        bs, bd, bat, bea, si, _ = (bufs[b][0], bufs[b][1], bufs[b][2],
                                   bufs[b][3], bufs[b][5], None)

        @pl.when(valid(t))
        def _():
            pltpu.make_async_copy(src_hbm.at[pl.ds(0, CH)], bs, si).wait()
            pltpu.make_async_copy(dst_hbm.at[pl.ds(0, CH)], bd, si).wait()
            pltpu.make_async_copy(attn_hbm.at[pl.ds(0, CH)], bat, si).wait()
            pltpu.make_async_copy(ea_hbm.at[pl.ds(0, CH)], bea, si).wait()

    def issue_gather(b, t):
        bs, _, _, _, bvr, _, sg = bufs[b]

        @pl.when(valid(t))
        def _():
            pltpu.async_copy(v_hbm.at[bs], bvr, sg)

    def wait_gather(b, t):
        _, _, _, _, bvr, _, sg = bufs[b]

        @pl.when(valid(t))
        def _():
            pltpu.make_async_copy(v_hbm.at[pl.ds(0, CH)], bvr, sg).wait()

    def compute(b, t):
        _, bd, bat, bea, bvr, _, _ = bufs[b]

        @pl.when(valid(t))
        def _():
            def grp(g, _):
                attnv = bat[pl.ds(g * 16, 16)]
                for l in range(16):
                    e = g * 16 + l
                    a = attnv[l]
                    for j in range(D // 16):
                        s16 = pl.ds(j * 16, 16)
                        bvr[e, s16] = bvr[e, s16] * a
                    bea[e, :] = bea[e, :] * a
                return 0

            lax.fori_loop(0, CH // 16, grp, 0)
            pltpu.sync_copy(bvr, outacc.at[bd], add=True)
            pltpu.sync_copy(bea, sacc.at[bd], add=True)

    issue_idx(0, 0)
    wait_idx(0, 0)
    issue_gather(0, 0)
    issue_idx(1, 1)

    def outer(t2, _):
        for b in (0, 1):
            t = t2 * 2 + b
            nb = 1 - b
            wait_idx(nb, t + 1)
            issue_gather(nb, t + 1)
            wait_gather(b, t)
            compute(b, t)
            issue_idx(b, t + 2)
        return 0

    lax.fori_loop(0, (CPW + 1) // 2, outer, 0)
    plsc.subcore_barrier()
    rs = pl.ds(s * SLC, SLC)
    pltpu.sync_copy(outacc.at[rs], outp_hbm.at[c, rs])
    pltpu.sync_copy(sacc.at[rs], sp_hbm.at[c, rs])


def _p4_body(a_hbm, b_hbm, src_hbm, dst_hbm, wp2_hbm, bp2_hbm,
             pred_hbm,
             is0, id0, ar0, br0, is1, id1, ar1, br1,
             out_v, wp2_v, bp2_v, si0, si1, sg0, sg1):
    bufs = ((is0, id0, ar0, br0, si0, sg0),
            (is1, id1, ar1, br1, si1, sg1))
    pltpu.sync_copy(wp2_hbm, wp2_v)
    pltpu.sync_copy(bp2_hbm, bp2_v)
    w = _wid()
    lane = lax.iota(jnp.int32, 16)
    bias = bp2_v[pl.ds(0, 16)][0]
    wp = [wp2_v[pl.ds(i * 16, 16)] for i in range(D // 16)]

    def valid(t):
        return (w + NW * t) < NCHUNKS

    def issue_idx(b, t):
        bs, bd, _, _, si, _ = bufs[b]
        base = (w + NW * t) * CH

        @pl.when(valid(t))
        def _():
            pltpu.async_copy(src_hbm.at[pl.ds(base, CH)], bs, si)
            pltpu.async_copy(dst_hbm.at[pl.ds(base, CH)], bd, si)

    def wait_idx(b, t):
        bs, bd, _, _, si, _ = bufs[b]

        @pl.when(valid(t))
        def _():
            pltpu.make_async_copy(src_hbm.at[pl.ds(0, CH)], bs, si).wait()
            pltpu.make_async_copy(dst_hbm.at[pl.ds(0, CH)], bd, si).wait()

    def issue_gather(b, t):
        bs, bd, ba, bb, _, sg = bufs[b]

        @pl.when(valid(t))
        def _():
            pltpu.async_copy(a_hbm.at[bs], ba, sg)
            pltpu.async_copy(b_hbm.at[bd], bb, sg)

    def wait_gather(b, t):
        _, _, ba, bb, _, sg = bufs[b]

        @pl.when(valid(t))
        def _():
            pltpu.make_async_copy(a_hbm.at[pl.ds(0, CH)], ba, sg).wait()
            pltpu.make_async_copy(b_hbm.at[pl.ds(0, CH)], bb, sg).wait()

    def compute(b, t):
        _, _, ba, bb, _, _ = bufs[b]

        @pl.when(valid(t))
        def _():
            base = (w + NW * t) * CH

            def grp(g, _):
                z = jnp.zeros((16,), jnp.float32)
                for l in range(16):
                    e = g * 16 + l
                    a16 = jnp.zeros((16,), jnp.float32)
                    for j in range(D // 16):
                        s16 = pl.ds(j * 16, 16)
                        h = jnp.maximum(ba[e, s16] + bb[e, s16], 0.0)
                        a16 = a16 + h * wp[j]
                    z = jnp.where(lane == l, jnp.sum(a16), z)
                z = z + bias
                out_v[pl.ds(g * 16, 16)] = 1.0 / (1.0 + jnp.exp(-z))
                return 0

            lax.fori_loop(0, CH // 16, grp, 0)
            pltpu.sync_copy(out_v, pred_hbm.at[pl.ds(base, CH)])

    issue_idx(0, 0)
    wait_idx(0, 0)
    issue_gather(0, 0)
    issue_idx(1, 1)

    def outer(t2, _):
        for b in (0, 1):
            t = t2 * 2 + b
            nb = 1 - b
            wait_idx(nb, t + 1)
            issue_gather(nb, t + 1)
            wait_gather(b, t)
            compute(b, t)
            issue_idx(b, t + 2)
        return 0

    lax.fori_loop(0, (CPW + 1) // 2, outer, 0)


_BR = 1000


def _tables_body(x_ref, wq, bq, wk, bk, wv, bv, we, ws, bs,
                 q_o, qe_o, k_o, v_o, skip_o):
    x = x_ref[...]
    q = (jnp.dot(x, wq[...], preferred_element_type=jnp.float32) + bq[...]) \
        * (1.0 / math.sqrt(C))
    q_o[...] = q
    qe_o[...] = lax.dot_general(q, we[...], (((1,), (1,)), ((), ())),
                                preferred_element_type=jnp.float32)
    k_o[...] = jnp.dot(x, wk[...], preferred_element_type=jnp.float32) + bk[...]
    v_o[...] = jnp.dot(x, wv[...], preferred_element_type=jnp.float32) + bv[...]
    skip_o[...] = jnp.dot(x, ws[...], preferred_element_type=jnp.float32) + bs[...]


def _combine_body(outp_ref, sp_ref, skip_ref, wed_ref, x_o):
    o = outp_ref[0] + outp_ref[1]
    sacc = sp_ref[0] + sp_ref[1]
    x_o[...] = o + jnp.dot(sacc, wed_ref[...],
                           preferred_element_type=jnp.float32) + skip_ref[...]


def _pred_tables_body(x2_ref, wp1_ref, bp1_ref, a_o, b_o):
    x2 = x2_ref[...]
    wp1 = wp1_ref[...]
    a_o[...] = jnp.dot(x2, wp1[0:C, :],
                       preferred_element_type=jnp.float32) + bp1_ref[...]
    b_o[...] = jnp.dot(x2, wp1[C:2 * C, :], preferred_element_type=jnp.float32)


def _full(shape):
    return pl.BlockSpec(shape, lambda i: tuple(0 for _ in shape))


def _tc_tables(x, wq, bq, wk, bk, wv, bv, we, ws, bs):
    f = jnp.float32
    return pl.pallas_call(
        _tables_body,
        grid=(N // _BR,),
        in_specs=[
            pl.BlockSpec((_BR, D), lambda i: (i, 0)),
            _full((D, C)), _full((1, C)),
            _full((D, C)), _full((1, C)),
            _full((D, C)), _full((1, C)),
            _full((DE, C)),
            _full((D, C)), _full((1, C)),
        ],
        out_specs=[
            pl.BlockSpec((_BR, C), lambda i: (i, 0)),
            pl.BlockSpec((_BR, DE), lambda i: (i, 0)),
            pl.BlockSpec((_BR, C), lambda i: (i, 0)),
            pl.BlockSpec((_BR, C), lambda i: (i, 0)),
            pl.BlockSpec((_BR, C), lambda i: (i, 0)),
        ],
        out_shape=[
            jax.ShapeDtypeStruct((N, C), f),
            jax.ShapeDtypeStruct((N, DE), f),
            jax.ShapeDtypeStruct((N, C), f),
            jax.ShapeDtypeStruct((N, C), f),
            jax.ShapeDtypeStruct((N, C), f),
        ],
    )(x, wq, bq, wk, bk, wv, bv, we, ws, bs)


def _tc_combine(outp, sp, skip, we_folded):
    return pl.pallas_call(
        _combine_body,
        grid=(N // _BR,),
        in_specs=[
            pl.BlockSpec((2, _BR, C), lambda i: (0, i, 0)),
            pl.BlockSpec((2, _BR, DE), lambda i: (0, i, 0)),
            pl.BlockSpec((_BR, C), lambda i: (i, 0)),
            _full((DE, C)),
        ],
        out_specs=pl.BlockSpec((_BR, C), lambda i: (i, 0)),
        out_shape=jax.ShapeDtypeStruct((N, C), jnp.float32),
    )(outp, sp, skip, we_folded)


def _tc_pred_tables(x2, wp1, bp1):
    return pl.pallas_call(
        _pred_tables_body,
        grid=(N // _BR,),
        in_specs=[
            pl.BlockSpec((_BR, C), lambda i: (i, 0)),
            _full((2 * C, C)),
            _full((1, C)),
        ],
        out_specs=[
            pl.BlockSpec((_BR, C), lambda i: (i, 0)),
            pl.BlockSpec((_BR, C), lambda i: (i, 0)),
        ],
        out_shape=[
            jax.ShapeDtypeStruct((N, C), jnp.float32),
            jax.ShapeDtypeStruct((N, C), jnp.float32),
        ],
    )(x2, wp1, bp1)


def _sc_p1(qext, k, ea, src, dst):
    f = jnp.float32
    return pl.kernel(
        _p1_body,
        out_type=[jax.ShapeDtypeStruct((E,), f),
                  jax.ShapeDtypeStruct((NW, NP), f)],
        mesh=_mesh,
        compiler_params=_sc_params,
        scratch_types=(
            [pltpu.VMEM((CH,), jnp.int32), pltpu.VMEM((CH,), jnp.int32),
             pltpu.VMEM((CH, DE), f), pltpu.VMEM((CH, D + DE), f),
             pltpu.VMEM((CH, D), f)] * 2 +
            [pltpu.VMEM((CH,), f), pltpu.VMEM((NP,), f)] +
            [pltpu.SemaphoreType.DMA] * 4),
    )(qext, k, ea, src, dst)


def _sc_p2(alpha, dst, amax):
    f = jnp.float32
    return pl.kernel(
        _p2_body,
        out_type=[jax.ShapeDtypeStruct((E,), f),
                  jax.ShapeDtypeStruct((NW, NP), f)],
        mesh=_mesh,
        compiler_params=_sc_params,
        scratch_types=[
            pltpu.VMEM((NP,), f),
            pltpu.VMEM((NP,), f),
            pltpu.VMEM((NP,), f),
            pltpu.VMEM((CH,), jnp.int32),
            pltpu.VMEM((CH,), f),
            pltpu.VMEM((CH,), f),
            pltpu.SemaphoreType.DMA,
        ],
    )(alpha, dst, amax)


def _sc_p2b(ex, dst, denom):
    f = jnp.float32
    return pl.kernel(
        _p2b_body,
        out_type=jax.ShapeDtypeStruct((E,), f),
        mesh=_mesh,
        compiler_params=_sc_params,
        scratch_types=[
            pltpu.VMEM((NP,), f),
            pltpu.VMEM((NP,), f),
            pltpu.VMEM((CH,), jnp.int32),
            pltpu.VMEM((CH,), f),
            pltpu.SemaphoreType.DMA,
        ],
    )(ex, dst, denom)


def _sc_p3(attn, src, dst, v, ea):
    f = jnp.float32
    return pl.kernel(
        _p3_body,
        out_type=[jax.ShapeDtypeStruct((NC, NP, C), f),
                  jax.ShapeDtypeStruct((NC, NP, DE), f)],
        mesh=_mesh,
        compiler_params=_sc_params,
        scratch_types=(
            [pltpu.VMEM((CH,), jnp.int32), pltpu.VMEM((CH,), jnp.int32),
             pltpu.VMEM((CH,), f), pltpu.VMEM((CH, DE), f),
             pltpu.VMEM((CH, C), f)] * 2 +
            [pltpu.VMEM_SHARED((NP, C), f), pltpu.VMEM_SHARED((NP, DE), f)] +
            [pltpu.SemaphoreType.DMA] * 4),
    )(attn, src, dst, v, ea)


def _sc_p4(a_tab, b_tab, src, dst, wp2, bp2):
    f = jnp.float32
    return pl.kernel(
        _p4_body,
        out_type=jax.ShapeDtypeStruct((E,), f),
        mesh=_mesh,
        compiler_params=_sc_params,
        scratch_types=(
            [pltpu.VMEM((CH,), jnp.int32), pltpu.VMEM((CH,), jnp.int32),
             pltpu.VMEM((CH, C), f), pltpu.VMEM((CH, C), f)] * 2 +
            [pltpu.VMEM((CH,), f), pltpu.VMEM((C,), f), pltpu.VMEM((16,), f)] +
            [pltpu.SemaphoreType.DMA] * 4),
    )(a_tab, b_tab, src, dst, wp2, bp2)


def _layer(x_tabs, ea, src, dst, we_folded):
    qs, qe, k, v, skip = x_tabs
    qext = jnp.concatenate([qs, qe], axis=1)
    alpha, amax = _sc_p1(qext, k, ea, src, dst)
    ex, denom = _sc_p2(alpha, dst, amax)
    attn = _sc_p2b(ex, dst, denom)
    outp, sp = _sc_p3(attn, src, dst, v, ea)
    return _tc_combine(outp[:, :N, :], sp[:, :N, :], skip, we_folded)


def kernel(x, edge_index, edge_attr,
           Wq1, bq1, Wk1, bk1, Wv1, bv1, We1, Ws1, bs1,
           Wq2, bq2, Wk2, bk2, Wv2, bv2, We2, Ws2, bs2,
           Wp1, bp1, Wp2, bp2):
    f = jnp.float32
    src = edge_index[0]
    dst = edge_index[1]
    m2 = jnp.dot(We1, We2)

    tabs1 = _tc_tables(x, Wq1, bq1.reshape(1, C), Wk1, bk1.reshape(1, C),
                       Wv1, bv1.reshape(1, C), We1, Ws1, bs1.reshape(1, C))
    x1 = _layer(tabs1, edge_attr, src, dst, We1)

    tabs2 = _tc_tables(x1, Wq2, bq2.reshape(1, C), Wk2, bk2.reshape(1, C),
                       Wv2, bv2.reshape(1, C), m2, Ws2, bs2.reshape(1, C))
    x2 = _layer(tabs2, edge_attr, src, dst, m2)

    a_tab, b_tab = _tc_pred_tables(x2, Wp1, bp1.reshape(1, C))
    wp2pad = jnp.pad(bp2.astype(f), (0, 15))
    return _sc_p4(a_tab, b_tab, src, dst, Wp2.reshape(C), wp2pad)


# speedup vs baseline: 1.6774x; 1.0938x over previous
import functools
import math

import jax
import jax.numpy as jnp
from jax import lax
from jax.experimental import pallas as pl
from jax.experimental.pallas import tpu as pltpu
from jax.experimental.pallas import tpu_sc as plsc

N = 10000
E = 320000
D = 128
DE = 16
C = 128

NC = 2
NS = 16
NW = NC * NS
NP = 10240
SLC = NP // NS
CH = 128
NCHUNKS = E // CH
CPW = -(-NCHUNKS // NW)

_mesh = plsc.VectorSubcoreMesh(
    core_axis_name="c", subcore_axis_name="s", num_cores=NC, num_subcores=NS)
_sc_params = pltpu.CompilerParams(use_tc_tiling_on_sc=False,
                                  needs_layout_passes=False)

_NEG = -3.0e38


def _wid():
    return lax.axis_index("s") * NC + lax.axis_index("c")


def _fill_1d(ref, val):
    n = ref.shape[0]

    def body(i, _):
        ref[pl.ds(i * 16, 16)] = jnp.full((16,), val, ref.dtype)
        return 0

    lax.fori_loop(0, n // 16, body, 0)


def _fill_2d(ref, val):
    r, cc = ref.shape

    def body(i, _):
        for j in range(cc // 16):
            ref[i, pl.ds(j * 16, 16)] = jnp.full((16,), val, ref.dtype)
        return 0

    lax.fori_loop(0, r, body, 0)


def _merge32(part_hbm, tab_ref, tmp_ref, op):
    pltpu.sync_copy(part_hbm.at[0], tab_ref)

    def body(t, _):
        pltpu.sync_copy(part_hbm.at[t], tmp_ref)

        def inner(i, _):
            sl = pl.ds(i * 16, 16)
            tab_ref[sl] = op(tab_ref[sl], tmp_ref[sl])
            return 0

        lax.fori_loop(0, NP // 16, inner, 0)
        return 0

    lax.fori_loop(1, NW, body, 0)


def _p1_body(qext_hbm, k_hbm, ea_hbm, src_hbm, dst_hbm,
             alpha_hbm, amax_hbm,
             is0, id0, ea0, q0, k0, is1, id1, ea1, q1, k1,
             alpha_v, amax_priv, si0, si1, sg0, sg1):
    bufs = ((is0, id0, ea0, q0, k0, si0, sg0),
            (is1, id1, ea1, q1, k1, si1, sg1))
    _fill_1d(amax_priv, _NEG)
    w = _wid()
    lane = lax.iota(jnp.int32, 16)

    def valid(t):
        return (w + NW * t) < NCHUNKS

    def issue_idx(b, t):
        bs, bd, bea, _, _, si, _ = bufs[b]
        base = (w + NW * t) * CH

        @pl.when(valid(t))
        def _():
            pltpu.async_copy(src_hbm.at[pl.ds(base, CH)], bs, si)
            pltpu.async_copy(dst_hbm.at[pl.ds(base, CH)], bd, si)
            pltpu.async_copy(ea_hbm.at[pl.ds(base, CH)], bea, si)

    def wait_idx(b, t):
        bs, bd, bea, _, _, si, _ = bufs[b]

        @pl.when(valid(t))
        def _():
            pltpu.make_async_copy(src_hbm.at[pl.ds(0, CH)], bs, si).wait()
            pltpu.make_async_copy(dst_hbm.at[pl.ds(0, CH)], bd, si).wait()
            pltpu.make_async_copy(ea_hbm.at[pl.ds(0, CH)], bea, si).wait()

    def issue_gather(b, t):
        bs, bd, _, bq, bk, _, sg = bufs[b]

        @pl.when(valid(t))
        def _():
            pltpu.async_copy(qext_hbm.at[bd], bq, sg)
            pltpu.async_copy(k_hbm.at[bs], bk, sg)

    def wait_gather(b, t):
        _, _, _, bq, bk, _, sg = bufs[b]

        @pl.when(valid(t))
        def _():
            pltpu.make_async_copy(qext_hbm.at[pl.ds(0, CH)], bq, sg).wait()
            pltpu.make_async_copy(k_hbm.at[pl.ds(0, CH)], bk, sg).wait()

    def compute(b, t):
        bs, bd, bea, bq, bk, _, _ = bufs[b]

        @pl.when(valid(t))
        def _():
            base = (w + NW * t) * CH

            def grp(g, _):
                acc = jnp.zeros((16,), jnp.float32)
                for l in range(16):
                    e = g * 16 + l
                    a16 = bq[e, pl.ds(D, DE)] * bea[e, :]
                    for j in range(D // 16):
                        s16 = pl.ds(j * 16, 16)
                        a16 = a16 + bq[e, s16] * bk[e, s16]
                    acc = jnp.where(lane == l, jnp.sum(a16), acc)
                sl = pl.ds(g * 16, 16)
                alpha_v[sl] = acc
                dv = bd[sl]

                def retry(cs):
                    i, _ = cs
                    cur = plsc.load_gather(amax_priv, [dv])
                    plsc.store_scatter(amax_priv, [dv], jnp.maximum(cur, acc))
                    chk = plsc.load_gather(amax_priv, [dv])
                    return i + 1, jnp.any(chk < acc)

                lax.while_loop(lambda cs: jnp.logical_and(cs[1], cs[0] < 16),
                               retry, (jnp.int32(0), jnp.bool_(True)))
                return 0

            lax.fori_loop(0, CH // 16, grp, 0)
            pltpu.sync_copy(alpha_v, alpha_hbm.at[pl.ds(base, CH)])

    issue_idx(0, 0)
    wait_idx(0, 0)
    issue_gather(0, 0)
    issue_idx(1, 1)

    def outer(t2, _):
        for b in (0, 1):
            t = t2 * 2 + b
            nb = 1 - b
            wait_idx(nb, t + 1)
            issue_gather(nb, t + 1)
            wait_gather(b, t)
            compute(b, t)
            issue_idx(b, t + 2)
        return 0

    lax.fori_loop(0, (CPW + 1) // 2, outer, 0)
    pltpu.sync_copy(amax_priv, amax_hbm.at[w])


def _p2_body(alpha_hbm, dst_hbm, amax_hbm,
             ex_hbm, denom_hbm,
             amax_tab, tmp_tab, denom_priv,
             id0, al0, id1, al1, ex_v, si0, si1, sem):
    bufs = ((id0, al0, si0), (id1, al1, si1))
    w = _wid()
    _merge32(amax_hbm, amax_tab, tmp_tab, jnp.maximum)
    _fill_1d(denom_priv, 0.0)

    def valid(t):
        return (w + NW * t) < NCHUNKS

    def issue(b, t):
        bd, ba, si = bufs[b]
        base = (w + NW * t) * CH

        @pl.when(valid(t))
        def _():
            pltpu.async_copy(dst_hbm.at[pl.ds(base, CH)], bd, si)
            pltpu.async_copy(alpha_hbm.at[pl.ds(base, CH)], ba, si)

    def wait(b, t):
        bd, ba, si = bufs[b]

        @pl.when(valid(t))
        def _():
            pltpu.make_async_copy(dst_hbm.at[pl.ds(0, CH)], bd, si).wait()
            pltpu.make_async_copy(alpha_hbm.at[pl.ds(0, CH)], ba, si).wait()

    def compute(b, t):
        bd, ba, _ = bufs[b]

        @pl.when(valid(t))
        def _():
            base = (w + NW * t) * CH

            def grp(g, _):
                sl = pl.ds(g * 16, 16)
                dv = bd[sl]
                mx = plsc.load_gather(amax_tab, [dv])
                exv = jnp.exp(ba[sl] - mx)
                ex_v[sl] = exv
                plsc.addupdate_scatter(denom_priv, [dv], exv)
                return 0

            lax.fori_loop(0, CH // 16, grp, 0)
            pltpu.sync_copy(ex_v, ex_hbm.at[pl.ds(base, CH)])

    issue(0, 0)
    issue(1, 1)

    def outer(t2, _):
        for b in (0, 1):
            t = t2 * 2 + b
            wait(b, t)
            compute(b, t)
            issue(b, t + 2)
        return 0

    lax.fori_loop(0, (CPW + 1) // 2, outer, 0)
    pltpu.sync_copy(denom_priv, denom_hbm.at[w])


def _p2b_body(ex_hbm, dst_hbm, denom_hbm,
              attn_hbm,
              denom_tab, tmp_tab,
              id0, ex0, id1, ex1, si0, si1, sem):
    bufs = ((id0, ex0, si0), (id1, ex1, si1))
    w = _wid()
    _merge32(denom_hbm, denom_tab, tmp_tab, jnp.add)

    def valid(t):
        return (w + NW * t) < NCHUNKS

    def issue(b, t):
        bd, be, si = bufs[b]
        base = (w + NW * t) * CH

        @pl.when(valid(t))
        def _():
            pltpu.async_copy(dst_hbm.at[pl.ds(base, CH)], bd, si)
            pltpu.async_copy(ex_hbm.at[pl.ds(base, CH)], be, si)

    def wait(b, t):
        bd, be, si = bufs[b]

        @pl.when(valid(t))
        def _():
            pltpu.make_async_copy(dst_hbm.at[pl.ds(0, CH)], bd, si).wait()
            pltpu.make_async_copy(ex_hbm.at[pl.ds(0, CH)], be, si).wait()

    def compute(b, t):
        bd, be, _ = bufs[b]

        @pl.when(valid(t))
        def _():
            base = (w + NW * t) * CH

            def grp(g, _):
                sl = pl.ds(g * 16, 16)
                dv = bd[sl]
                den = plsc.load_gather(denom_tab, [dv])
                be[sl] = be[sl] / (den + 1e-16)
                return 0

            lax.fori_loop(0, CH // 16, grp, 0)
            pltpu.sync_copy(be, attn_hbm.at[pl.ds(base, CH)])

    issue(0, 0)
    issue(1, 1)

    def outer(t2, _):
        for b in (0, 1):
            t = t2 * 2 + b
            wait(b, t)
            compute(b, t)
            issue(b, t + 2)
        return 0

    lax.fori_loop(0, (CPW + 1) // 2, outer, 0)


def _p3_body(attn_hbm, src_hbm, dst_hbm, v_hbm, ea_hbm,
             outp_hbm, sp_hbm,
             is0, id0, at0, ea0, vr0, is1, id1, at1, ea1, vr1,
             outacc, sacc, si0, si1, sg0, sg1):
    bufs = ((is0, id0, at0, ea0, vr0, si0, sg0),
            (is1, id1, at1, ea1, vr1, si1, sg1))
    c = lax.axis_index("c")
    s = lax.axis_index("s")
    w = _wid()
    _fill_2d(vr0, 0.0)
    _fill_2d(ea0, 0.0)
    for i in range(SLC // CH):
        rs = pl.ds(s * SLC + i * CH, CH)
        pltpu.sync_copy(vr0, outacc.at[rs])
        pltpu.sync_copy(ea0, sacc.at[rs])
    plsc.subcore_barrier()

    def valid(t):
        return (w + NW * t) < NCHUNKS

    def issue_idx(b, t):
        bs, bd, bat, bea, _, si, _ = bufs[b]
        base = (w + NW * t) * CH

        @pl.when(valid(t))
        def _():
            pltpu.async_copy(src_hbm.at[pl.ds(base, CH)], bs, si)
            pltpu.async_copy(dst_hbm.at[pl.ds(base, CH)], bd, si)
            pltpu.async_copy(attn_hbm.at[pl.ds(base, CH)], bat, si)
            pltpu.async_copy(ea_hbm.at[pl.ds(base, CH)], bea, si)

    def wait_idx(b, t):
        bs, bd, bat, bea, si, _ = (bufs[b][0], bufs[b][1], bufs[b][2],
                                   bufs[b][3], bufs[b][5], None)

        @pl.when(valid(t))
        def _():
            pltpu.make_async_copy(src_hbm.at[pl.ds(0, CH)], bs, si).wait()
            pltpu.make_async_copy(dst_hbm.at[pl.ds(0, CH)], bd, si).wait()
            pltpu.make_async_copy(attn_hbm.at[pl.ds(0, CH)], bat, si).wait()
            pltpu.make_async_copy(ea_hbm.at[pl.ds(0, CH)], bea, si).wait()

    def issue_gather(b, t):
        bs, _, _, _, bvr, _, sg = bufs[b]

        @pl.when(valid(t))
        def _():
            pltpu.async_copy(v_hbm.at[bs], bvr, sg)

    def wait_gather(b, t):
        _, _, _, _, bvr, _, sg = bufs[b]

        @pl.when(valid(t))
        def _():
            pltpu.make_async_copy(v_hbm.at[pl.ds(0, CH)], bvr, sg).wait()

    def compute(b, t):
        _, bd, bat, bea, bvr, _, _ = bufs[b]

        @pl.when(valid(t))
        def _():
            def grp(g, _):
                attnv = bat[pl.ds(g * 16, 16)]
                for l in range(16):
                    e = g * 16 + l
                    a = attnv[l]
                    for j in range(D // 16):
                        s16 = pl.ds(j * 16, 16)
                        bvr[e, s16] = bvr[e, s16] * a
                    bea[e, :] = bea[e, :] * a
                return 0

            lax.fori_loop(0, CH // 16, grp, 0)
            pltpu.sync_copy(bvr, outacc.at[bd], add=True)
            pltpu.sync_copy(bea, sacc.at[bd], add=True)

    issue_idx(0, 0)
    wait_idx(0, 0)
    issue_gather(0, 0)
    issue_idx(1, 1)

    def outer(t2, _):
        for b in (0, 1):
            t = t2 * 2 + b
            nb = 1 - b
            wait_idx(nb, t + 1)
            issue_gather(nb, t + 1)
            wait_gather(b, t)
            compute(b, t)
            issue_idx(b, t + 2)
        return 0

    lax.fori_loop(0, (CPW + 1) // 2, outer, 0)
    plsc.subcore_barrier()
    rs = pl.ds(s * SLC, SLC)
    pltpu.sync_copy(outacc.at[rs], outp_hbm.at[c, rs])
    pltpu.sync_copy(sacc.at[rs], sp_hbm.at[c, rs])


def _p4_body(a_hbm, b_hbm, src_hbm, dst_hbm, wp2_hbm, bp2_hbm,
             pred_hbm,
             is0, id0, ar0, br0, is1, id1, ar1, br1,
             out_v, wp2_v, bp2_v, si0, si1, sg0, sg1):
    bufs = ((is0, id0, ar0, br0, si0, sg0),
            (is1, id1, ar1, br1, si1, sg1))
    pltpu.sync_copy(wp2_hbm, wp2_v)
    pltpu.sync_copy(bp2_hbm, bp2_v)
    w = _wid()
    lane = lax.iota(jnp.int32, 16)
    bias = bp2_v[pl.ds(0, 16)][0]
    wp = [wp2_v[pl.ds(i * 16, 16)] for i in range(D // 16)]

    def valid(t):
        return (w + NW * t) < NCHUNKS

    def issue_idx(b, t):
        bs, bd, _, _, si, _ = bufs[b]
        base = (w + NW * t) * CH

        @pl.when(valid(t))
        def _():
            pltpu.async_copy(src_hbm.at[pl.ds(base, CH)], bs, si)
            pltpu.async_copy(dst_hbm.at[pl.ds(base, CH)], bd, si)

    def wait_idx(b, t):
        bs, bd, _, _, si, _ = bufs[b]

        @pl.when(valid(t))
        def _():
            pltpu.make_async_copy(src_hbm.at[pl.ds(0, CH)], bs, si).wait()
            pltpu.make_async_copy(dst_hbm.at[pl.ds(0, CH)], bd, si).wait()

    def issue_gather(b, t):
        bs, bd, ba, bb, _, sg = bufs[b]

        @pl.when(valid(t))
        def _():
            pltpu.async_copy(a_hbm.at[bs], ba, sg)
            pltpu.async_copy(b_hbm.at[bd], bb, sg)

    def wait_gather(b, t):
        _, _, ba, bb, _, sg = bufs[b]

        @pl.when(valid(t))
        def _():
            pltpu.make_async_copy(a_hbm.at[pl.ds(0, CH)], ba, sg).wait()
            pltpu.make_async_copy(b_hbm.at[pl.ds(0, CH)], bb, sg).wait()

    def compute(b, t):
        _, _, ba, bb, _, _ = bufs[b]

        @pl.when(valid(t))
        def _():
            base = (w + NW * t) * CH

            def grp(g, _):
                z = jnp.zeros((16,), jnp.float32)
                for l in range(16):
                    e = g * 16 + l
                    a16 = jnp.zeros((16,), jnp.float32)
                    for j in range(D // 16):
                        s16 = pl.ds(j * 16, 16)
                        h = jnp.maximum(ba[e, s16] + bb[e, s16], 0.0)
                        a16 = a16 + h * wp[j]
                    z = jnp.where(lane == l, jnp.sum(a16), z)
                z = z + bias
                out_v[pl.ds(g * 16, 16)] = 1.0 / (1.0 + jnp.exp(-z))
                return 0

            lax.fori_loop(0, CH // 16, grp, 0)
            pltpu.sync_copy(out_v, pred_hbm.at[pl.ds(base, CH)])

    issue_idx(0, 0)
    wait_idx(0, 0)
    issue_gather(0, 0)
    issue_idx(1, 1)

    def outer(t2, _):
        for b in (0, 1):
            t = t2 * 2 + b
            nb = 1 - b
            wait_idx(nb, t + 1)
            issue_gather(nb, t + 1)
            wait_gather(b, t)
            compute(b, t)
            issue_idx(b, t + 2)
        return 0

    lax.fori_loop(0, (CPW + 1) // 2, outer, 0)


_BR = 1000


def _tables_body(x_ref, wq, bq, wk, bk, wv, bv, we, ws, bs,
                 q_o, qe_o, k_o, v_o, skip_o):
    x = x_ref[...]
    q = (jnp.dot(x, wq[...], preferred_element_type=jnp.float32) + bq[...]) \
        * (1.0 / math.sqrt(C))
    q_o[...] = q
    qe_o[...] = lax.dot_general(q, we[...], (((1,), (1,)), ((), ())),
                                preferred_element_type=jnp.float32)
    k_o[...] = jnp.dot(x, wk[...], preferred_element_type=jnp.float32) + bk[...]
    v_o[...] = jnp.dot(x, wv[...], preferred_element_type=jnp.float32) + bv[...]
    skip_o[...] = jnp.dot(x, ws[...], preferred_element_type=jnp.float32) + bs[...]


def _combine_body(outp_ref, sp_ref, skip_ref, wed_ref, x_o):
    o = outp_ref[0] + outp_ref[1]
    sacc = sp_ref[0] + sp_ref[1]
    x_o[...] = o + jnp.dot(sacc, wed_ref[...],
                           preferred_element_type=jnp.float32) + skip_ref[...]


def _pred_tables_body(x2_ref, wp1_ref, bp1_ref, a_o, b_o):
    x2 = x2_ref[...]
    wp1 = wp1_ref[...]
    a_o[...] = jnp.dot(x2, wp1[0:C, :],
                       preferred_element_type=jnp.float32) + bp1_ref[...]
    b_o[...] = jnp.dot(x2, wp1[C:2 * C, :], preferred_element_type=jnp.float32)


def _full(shape):
    return pl.BlockSpec(shape, lambda i: tuple(0 for _ in shape))


def _tc_tables(x, wq, bq, wk, bk, wv, bv, we, ws, bs):
    f = jnp.float32
    return pl.pallas_call(
        _tables_body,
        grid=(N // _BR,),
        in_specs=[
            pl.BlockSpec((_BR, D), lambda i: (i, 0)),
            _full((D, C)), _full((1, C)),
            _full((D, C)), _full((1, C)),
            _full((D, C)), _full((1, C)),
            _full((DE, C)),
            _full((D, C)), _full((1, C)),
        ],
        out_specs=[
            pl.BlockSpec((_BR, C), lambda i: (i, 0)),
            pl.BlockSpec((_BR, DE), lambda i: (i, 0)),
            pl.BlockSpec((_BR, C), lambda i: (i, 0)),
            pl.BlockSpec((_BR, C), lambda i: (i, 0)),
            pl.BlockSpec((_BR, C), lambda i: (i, 0)),
        ],
        out_shape=[
            jax.ShapeDtypeStruct((N, C), f),
            jax.ShapeDtypeStruct((N, DE), f),
            jax.ShapeDtypeStruct((N, C), f),
            jax.ShapeDtypeStruct((N, C), f),
            jax.ShapeDtypeStruct((N, C), f),
        ],
    )(x, wq, bq, wk, bk, wv, bv, we, ws, bs)


def _tc_combine(outp, sp, skip, we_folded):
    return pl.pallas_call(
        _combine_body,
        grid=(N // _BR,),
        in_specs=[
            pl.BlockSpec((2, _BR, C), lambda i: (0, i, 0)),
            pl.BlockSpec((2, _BR, DE), lambda i: (0, i, 0)),
            pl.BlockSpec((_BR, C), lambda i: (i, 0)),
            _full((DE, C)),
        ],
        out_specs=pl.BlockSpec((_BR, C), lambda i: (i, 0)),
        out_shape=jax.ShapeDtypeStruct((N, C), jnp.float32),
    )(outp, sp, skip, we_folded)


def _tc_pred_tables(x2, wp1, bp1):
    return pl.pallas_call(
        _pred_tables_body,
        grid=(N // _BR,),
        in_specs=[
            pl.BlockSpec((_BR, C), lambda i: (i, 0)),
            _full((2 * C, C)),
            _full((1, C)),
        ],
        out_specs=[
            pl.BlockSpec((_BR, C), lambda i: (i, 0)),
            pl.BlockSpec((_BR, C), lambda i: (i, 0)),
        ],
        out_shape=[
            jax.ShapeDtypeStruct((N, C), jnp.float32),
            jax.ShapeDtypeStruct((N, C), jnp.float32),
        ],
    )(x2, wp1, bp1)


def _sc_p1(qext, k, ea, src, dst):
    f = jnp.float32
    return pl.kernel(
        _p1_body,
        out_type=[jax.ShapeDtypeStruct((E,), f),
                  jax.ShapeDtypeStruct((NW, NP), f)],
        mesh=_mesh,
        compiler_params=_sc_params,
        scratch_types=(
            [pltpu.VMEM((CH,), jnp.int32), pltpu.VMEM((CH,), jnp.int32),
             pltpu.VMEM((CH, DE), f), pltpu.VMEM((CH, D + DE), f),
             pltpu.VMEM((CH, D), f)] * 2 +
            [pltpu.VMEM((CH,), f), pltpu.VMEM((NP,), f)] +
            [pltpu.SemaphoreType.DMA] * 4),
    )(qext, k, ea, src, dst)


def _sc_p2(alpha, dst, amax):
    f = jnp.float32
    return pl.kernel(
        _p2_body,
        out_type=[jax.ShapeDtypeStruct((E,), f),
                  jax.ShapeDtypeStruct((NW, NP), f)],
        mesh=_mesh,
        compiler_params=_sc_params,
        scratch_types=(
            [pltpu.VMEM((NP,), f)] * 3 +
            [pltpu.VMEM((CH,), jnp.int32), pltpu.VMEM((CH,), f)] * 2 +
            [pltpu.VMEM((CH,), f)] +
            [pltpu.SemaphoreType.DMA] * 3),
    )(alpha, dst, amax)


def _sc_p2b(ex, dst, denom):
    f = jnp.float32
    return pl.kernel(
        _p2b_body,
        out_type=jax.ShapeDtypeStruct((E,), f),
        mesh=_mesh,
        compiler_params=_sc_params,
        scratch_types=(
            [pltpu.VMEM((NP,), f)] * 2 +
            [pltpu.VMEM((CH,), jnp.int32), pltpu.VMEM((CH,), f)] * 2 +
            [pltpu.SemaphoreType.DMA] * 3),
    )(ex, dst, denom)


def _sc_p3(attn, src, dst, v, ea):
    f = jnp.float32
    return pl.kernel(
        _p3_body,
        out_type=[jax.ShapeDtypeStruct((NC, NP, C), f),
                  jax.ShapeDtypeStruct((NC, NP, DE), f)],
        mesh=_mesh,
        compiler_params=_sc_params,
        scratch_types=(
            [pltpu.VMEM((CH,), jnp.int32), pltpu.VMEM((CH,), jnp.int32),
             pltpu.VMEM((CH,), f), pltpu.VMEM((CH, DE), f),
             pltpu.VMEM((CH, C), f)] * 2 +
            [pltpu.VMEM_SHARED((NP, C), f), pltpu.VMEM_SHARED((NP, DE), f)] +
            [pltpu.SemaphoreType.DMA] * 4),
    )(attn, src, dst, v, ea)


def _sc_p4(a_tab, b_tab, src, dst, wp2, bp2):
    f = jnp.float32
    return pl.kernel(
        _p4_body,
        out_type=jax.ShapeDtypeStruct((E,), f),
        mesh=_mesh,
        compiler_params=_sc_params,
        scratch_types=(
            [pltpu.VMEM((CH,), jnp.int32), pltpu.VMEM((CH,), jnp.int32),
             pltpu.VMEM((CH, C), f), pltpu.VMEM((CH, C), f)] * 2 +
            [pltpu.VMEM((CH,), f), pltpu.VMEM((C,), f), pltpu.VMEM((16,), f)] +
            [pltpu.SemaphoreType.DMA] * 4),
    )(a_tab, b_tab, src, dst, wp2, bp2)


def _layer(x_tabs, ea, src, dst, we_folded):
    qs, qe, k, v, skip = x_tabs
    qext = jnp.concatenate([qs, qe], axis=1)
    alpha, amax = _sc_p1(qext, k, ea, src, dst)
    ex, denom = _sc_p2(alpha, dst, amax)
    attn = _sc_p2b(ex, dst, denom)
    outp, sp = _sc_p3(attn, src, dst, v, ea)
    return _tc_combine(outp[:, :N, :], sp[:, :N, :], skip, we_folded)


def kernel(x, edge_index, edge_attr,
           Wq1, bq1, Wk1, bk1, Wv1, bv1, We1, Ws1, bs1,
           Wq2, bq2, Wk2, bk2, Wv2, bv2, We2, Ws2, bs2,
           Wp1, bp1, Wp2, bp2):
    f = jnp.float32
    src = edge_index[0]
    dst = edge_index[1]
    m2 = jnp.dot(We1, We2)

    tabs1 = _tc_tables(x, Wq1, bq1.reshape(1, C), Wk1, bk1.reshape(1, C),
                       Wv1, bv1.reshape(1, C), We1, Ws1, bs1.reshape(1, C))
    x1 = _layer(tabs1, edge_attr, src, dst, We1)

    tabs2 = _tc_tables(x1, Wq2, bq2.reshape(1, C), Wk2, bk2.reshape(1, C),
                       Wv2, bv2.reshape(1, C), m2, Ws2, bs2.reshape(1, C))
    x2 = _layer(tabs2, edge_attr, src, dst, m2)

    a_tab, b_tab = _tc_pred_tables(x2, Wp1, bp1.reshape(1, C))
    wp2pad = jnp.pad(bp2.astype(f), (0, 15))
    return _sc_p4(a_tab, b_tab, src, dst, Wp2.reshape(C), wp2pad)
